# Initial kernel scaffold; baseline (speedup 1.0000x reference)
import jax
import jax.numpy as jnp
from jax.experimental import pallas as pl


def kernel(x, edge_index, W_enc, b_enc, W_g1, b_g1, W_g2, b_g2, W_sw, b_sw, W_v, b_v, cvx_p_inj, cvx_q_inj, cvx_y0, cvx_r_pu, cvx_x_pu, cvx_bigM_flow, cvx_bigM_v, cvx_A_from, cvx_A_to, cvx_sub_mask, cvx_non_sub_mask, cvx_bigM_flow_sq, cvx_z_line_sq):
    raise NotImplementedError("write your pallas kernel here")


# speedup vs baseline: 8.2297x; 8.2297x over previous
import jax
import jax.numpy as jnp
from jax import lax
from jax.experimental import pallas as pl

_N = 1000
_E = 1200
_DIN = 128
_H = 256
_L = 128

_f32 = jnp.float32
_bf16 = jnp.bfloat16


def _split(a):
    ah = a.astype(_bf16)
    al = (a - ah.astype(_f32)).astype(_bf16)
    return ah, al


def _dot3(a, b):
    ah, al = _split(a)
    bh, bl = _split(b)
    hh = jnp.dot(ah, bh, preferred_element_type=_f32)
    hl = jnp.dot(ah, bl, preferred_element_type=_f32)
    lh = jnp.dot(al, bh, preferred_element_type=_f32)
    return hh + (hl + lh)


def _gnn_body(x_ref, src_row_ref, dst_row_ref, src_col_ref, dst_col_ref,
              W_enc_ref, b_enc_ref, W_g1_ref, b_g1_ref, W_g2_ref, b_g2_ref,
              w_head_ref, b_sw_ref, b_v_ref,
              yw_ref, vw_ref):
    dst_row = dst_row_ref[...]
    src_col = src_col_ref[...]
    dst_col = dst_col_ref[...]

    ioNE = lax.broadcasted_iota(jnp.int32, (_N, _E), 0)
    ohT_dst_f = (ioNE == dst_row).astype(_f32)
    ohT_dst = ohT_dst_f.astype(_bf16)
    ioEN = lax.broadcasted_iota(jnp.int32, (_E, _N), 1)
    oh_src = (ioEN == src_col).astype(_f32).astype(_bf16)
    oh_dst = (ioEN == dst_col).astype(_f32).astype(_bf16)

    deg = jnp.sum(ohT_dst_f, axis=1, keepdims=True) + 1.0
    dinv = lax.rsqrt(jnp.maximum(deg, 1.0))

    M = jnp.dot(ohT_dst, oh_src, preferred_element_type=_f32)
    M_bf = M.astype(_bf16)

    def conv(t):
        g = dinv * t
        gh, gl = _split(g)
        agg = (jnp.dot(M_bf, gh, preferred_element_type=_f32)
               + jnp.dot(M_bf, gl, preferred_element_type=_f32))
        return dinv * (agg + g)

    x = x_ref[...]
    h0 = jax.nn.relu(_dot3(x, W_enc_ref[...]) + b_enc_ref[...])
    t1 = _dot3(h0, W_g1_ref[...])
    h1 = jax.nn.relu(conv(t1) + b_g1_ref[...])
    t2 = _dot3(h1, W_g2_ref[...])
    h2 = jax.nn.relu(conv(t2) + b_g2_ref[...])

    sv = jnp.dot(h2.astype(_bf16), w_head_ref[...].astype(_bf16),
                 preferred_element_type=_f32)
    s_src = sv[:, 0:1].astype(_bf16)
    s_dst = sv[:, 1:2].astype(_bf16)
    e_src = jnp.dot(oh_src, s_src, preferred_element_type=_f32)
    e_dst = jnp.dot(oh_dst, s_dst, preferred_element_type=_f32)
    yw_ref[...] = jax.nn.sigmoid(e_src + e_dst + b_sw_ref[...])

    vr = jax.nn.sigmoid(sv[:, 2:3] + b_v_ref[...])
    vw_ref[...] = (0.9 + 0.2 * vr) ** 2


def kernel(x, edge_index, W_enc, b_enc, W_g1, b_g1, W_g2, b_g2, W_sw, b_sw,
           W_v, b_v, cvx_p_inj, cvx_q_inj, cvx_y0, cvx_r_pu, cvx_x_pu,
           cvx_bigM_flow, cvx_bigM_v, cvx_A_from, cvx_A_to, cvx_sub_mask,
           cvx_non_sub_mask, cvx_bigM_flow_sq, cvx_z_line_sq):
    src = edge_index[0]
    dst = edge_index[1]
    yw2, vw2 = pl.pallas_call(
        _gnn_body,
        out_shape=[
            jax.ShapeDtypeStruct((_E, 1), _f32),
            jax.ShapeDtypeStruct((_N, 1), _f32),
        ],
    )(x,
      src.reshape(1, _E), dst.reshape(1, _E),
      src.reshape(_E, 1), dst.reshape(_E, 1),
      W_enc, b_enc.reshape(1, _H),
      W_g1, b_g1.reshape(1, _H),
      W_g2, b_g2.reshape(1, _L),
      jnp.concatenate([W_sw[:_L], W_sw[_L:], W_v], axis=1),
      b_sw.reshape(1, 1), b_v.reshape(1, 1))
    return yw2[:, 0], vw2[:, 0]
